# baseline (speedup 1.0000x reference)
import jax
import jax.numpy as jnp
from jax.experimental import pallas as pl

_TOKENS = 32768
_DIM = 768
_NUM_EXPERTS = 8
_BLOCK = 2048


def _router_block(x_ref, wt_ref, b_ref, gates_out_ref, idx_out_ref):
    x = x_ref[...]
    wt = wt_ref[...]
    b = b_ref[...]
    logits = jnp.dot(x, wt, preferred_element_type=jnp.float32) + b

    m = jnp.max(logits, axis=-1, keepdims=True)
    e = jnp.exp(logits - m)
    s = jnp.sum(e, axis=-1, keepdims=True)
    gates = e / s

    shape = gates.shape
    iota = jax.lax.broadcasted_iota(jnp.int32, shape, 1)

    m1 = jnp.max(gates, axis=-1, keepdims=True)
    i1 = jnp.min(jnp.where(gates == m1, iota, _NUM_EXPERTS), axis=-1,
                 keepdims=True)
    masked = jnp.where(iota == i1, -jnp.inf, gates)
    m2 = jnp.max(masked, axis=-1, keepdims=True)
    i2 = jnp.min(jnp.where(masked == m2, iota, _NUM_EXPERTS), axis=-1,
                 keepdims=True)

    gates_out_ref[...] = jnp.concatenate([m1, m2], axis=-1)
    idx_out_ref[...] = jnp.concatenate([i1, i2], axis=-1)


def kernel(x, W, b):
    wt = W.T
    b2 = b.reshape(1, _NUM_EXPERTS)
    grid = (_TOKENS // _BLOCK,)
    out = pl.pallas_call(
        _router_block,
        grid=grid,
        in_specs=[
            pl.BlockSpec((_BLOCK, _DIM), lambda i: (i, 0)),
            pl.BlockSpec((_DIM, _NUM_EXPERTS), lambda i: (0, 0)),
            pl.BlockSpec((1, _NUM_EXPERTS), lambda i: (0, 0)),
        ],
        out_specs=[
            pl.BlockSpec((_BLOCK, 2), lambda i: (i, 0)),
            pl.BlockSpec((_BLOCK, 2), lambda i: (i, 0)),
        ],
        out_shape=[
            jax.ShapeDtypeStruct((_TOKENS, 2), jnp.float32),
            jax.ShapeDtypeStruct((_TOKENS, 2), jnp.int32),
        ],
    )(x, wt, b2)
    return (out[0], out[1])


# speedup vs baseline: 2.1208x; 2.1208x over previous
import jax
import jax.numpy as jnp
from jax.experimental import pallas as pl

_TOKENS = 32768
_DIM = 768
_NUM_EXPERTS = 8
_BLOCK = 2048


def _router_block(x_ref, w_ref, b_ref, gates_out_ref, idx_out_ref):
    x = x_ref[...]
    w = w_ref[...]
    b = b_ref[...]
    logits = jax.lax.dot_general(
        w, x, (((1,), (1,)), ((), ())),
        preferred_element_type=jnp.float32) + b

    m = jnp.max(logits, axis=0, keepdims=True)
    e = jnp.exp(logits - m)
    s = jnp.sum(e, axis=0, keepdims=True)
    gates = e / s

    iota = jax.lax.broadcasted_iota(jnp.int32, gates.shape, 0)

    m1 = jnp.max(gates, axis=0, keepdims=True)
    i1 = jnp.min(jnp.where(gates == m1, iota, _NUM_EXPERTS), axis=0,
                 keepdims=True)
    masked = jnp.where(iota == i1, -jnp.inf, gates)
    m2 = jnp.max(masked, axis=0, keepdims=True)
    i2 = jnp.min(jnp.where(masked == m2, iota, _NUM_EXPERTS), axis=0,
                 keepdims=True)

    gates_out_ref[...] = jnp.concatenate([m1, m2], axis=0)
    idx_out_ref[...] = jnp.concatenate([i1, i2], axis=0)


def kernel(x, W, b):
    b2 = b.reshape(_NUM_EXPERTS, 1)
    grid = (_TOKENS // _BLOCK,)
    out = pl.pallas_call(
        _router_block,
        grid=grid,
        in_specs=[
            pl.BlockSpec((_BLOCK, _DIM), lambda i: (i, 0)),
            pl.BlockSpec((_NUM_EXPERTS, _DIM), lambda i: (0, 0)),
            pl.BlockSpec((_NUM_EXPERTS, 1), lambda i: (0, 0)),
        ],
        out_specs=[
            pl.BlockSpec((2, _BLOCK), lambda i: (0, i)),
            pl.BlockSpec((2, _BLOCK), lambda i: (0, i)),
        ],
        out_shape=[
            jax.ShapeDtypeStruct((2, _TOKENS), jnp.float32),
            jax.ShapeDtypeStruct((2, _TOKENS), jnp.int32),
        ],
    )(x, W, b2)
    return (out[0].T, out[1].T)


# speedup vs baseline: 2.2608x; 1.0660x over previous
import jax
import jax.numpy as jnp
from jax.experimental import pallas as pl

_TOKENS = 32768
_DIM = 768
_NUM_EXPERTS = 8
_BLOCK = 4096


def _router_block(x_ref, w_ref, b_ref, gates_out_ref, idx_out_ref):
    x = x_ref[...]
    w = w_ref[...]
    b = b_ref[...]
    logits = jax.lax.dot_general(
        w, x, (((1,), (1,)), ((), ())),
        preferred_element_type=jnp.float32) + b

    m = jnp.max(logits, axis=0, keepdims=True)
    e = jnp.exp(logits - m)
    s = jnp.sum(e, axis=0, keepdims=True)
    gates = e / s

    iota = jax.lax.broadcasted_iota(jnp.int32, gates.shape, 0)

    m1 = jnp.max(gates, axis=0, keepdims=True)
    i1 = jnp.min(jnp.where(gates == m1, iota, _NUM_EXPERTS), axis=0,
                 keepdims=True)
    masked = jnp.where(iota == i1, -jnp.inf, gates)
    m2 = jnp.max(masked, axis=0, keepdims=True)
    i2 = jnp.min(jnp.where(masked == m2, iota, _NUM_EXPERTS), axis=0,
                 keepdims=True)

    gates_out_ref[...] = jnp.concatenate([m1, m2], axis=0)
    idx_out_ref[...] = jnp.concatenate([i1, i2], axis=0)


def kernel(x, W, b):
    b2 = b.reshape(_NUM_EXPERTS, 1)
    grid = (_TOKENS // _BLOCK,)
    out = pl.pallas_call(
        _router_block,
        grid=grid,
        in_specs=[
            pl.BlockSpec((_BLOCK, _DIM), lambda i: (i, 0)),
            pl.BlockSpec((_NUM_EXPERTS, _DIM), lambda i: (0, 0)),
            pl.BlockSpec((_NUM_EXPERTS, 1), lambda i: (0, 0)),
        ],
        out_specs=[
            pl.BlockSpec((2, _BLOCK), lambda i: (0, i)),
            pl.BlockSpec((2, _BLOCK), lambda i: (0, i)),
        ],
        out_shape=[
            jax.ShapeDtypeStruct((2, _TOKENS), jnp.float32),
            jax.ShapeDtypeStruct((2, _TOKENS), jnp.int32),
        ],
    )(x, W, b2)
    return (out[0].T, out[1].T)
